# baseline (speedup 1.0000x reference)
import functools

import jax
import jax.numpy as jnp
from jax import lax
from jax.experimental import pallas as pl
from jax.experimental.pallas import tpu as pltpu
from jax.experimental.pallas import tpu_sc as plsc

N = 320000
D = 128
NUM_SEGMENTS = 10000

NUM_CORES = 2
NUM_SUBCORES = 16
NUM_WORKERS = NUM_CORES * NUM_SUBCORES

ROWS_PER_WORKER = N // NUM_WORKERS
CHUNK = 128
NUM_FULL_CHUNKS = ROWS_PER_WORKER // CHUNK
TAIL = ROWS_PER_WORKER - NUM_FULL_CHUNKS * CHUNK
STRIP = 624
STRIP_COVERED = STRIP * NUM_SUBCORES
STRIP_REM = NUM_SEGMENTS - STRIP_COVERED


def _seg_sum_sc(x, batch_index):
    mesh = plsc.VectorSubcoreMesh(core_axis_name="c", subcore_axis_name="s")

    @functools.partial(
        pl.kernel,
        mesh=mesh,
        out_type=jax.ShapeDtypeStruct((NUM_CORES, NUM_SEGMENTS, D), jnp.float32),
        scratch_types=[
            pltpu.VMEM_SHARED((NUM_SEGMENTS, D), jnp.float32),
            pltpu.VMEM((CHUNK, D), jnp.float32),
            pltpu.VMEM((CHUNK,), jnp.int32),
            pltpu.VMEM((CHUNK, D), jnp.float32),
            pltpu.VMEM((CHUNK,), jnp.int32),
            pltpu.VMEM((CHUNK, D), jnp.float32),
            pltpu.VMEM((CHUNK,), jnp.int32),
            pltpu.VMEM((TAIL,), jnp.int32),
            pltpu.SemaphoreType.DMA,
            pltpu.SemaphoreType.DMA,
            pltpu.SemaphoreType.DMA,
            pltpu.SemaphoreType.DMA,
            pltpu.SemaphoreType.DMA,
            pltpu.SemaphoreType.DMA,
            pltpu.SemaphoreType.DMA,
            pltpu.SemaphoreType.DMA,
            pltpu.SemaphoreType.DMA,
        ],
    )
    def k(x_hbm, idx_hbm, out_hbm, acc, rows_0, idx_0, rows_1, idx_1,
          rows_2, idx_2, idx_t, sr0, si0, sr1, si1, sr2, si2,
          ss0, ss1, ss2):
        c = lax.axis_index("c")
        s = lax.axis_index("s")
        wid = c * NUM_SUBCORES + s
        base = wid * ROWS_PER_WORKER

        bufs = (
            (rows_0, idx_0, sr0, si0, ss0),
            (rows_1, idx_1, sr1, si1, ss1),
            (rows_2, idx_2, sr2, si2, ss2),
        )

        def load_chunk(i, b):
            rb, ib, sr, si, _ = bufs[b]
            row0 = base + i * CHUNK
            pltpu.async_copy(x_hbm.at[pl.ds(row0, CHUNK)], rb, sr)
            pltpu.async_copy(idx_hbm.at[pl.ds(row0, CHUNK)], ib, si)

        def wait_load(i, b):
            rb, ib, sr, si, _ = bufs[b]
            row0 = base + i * CHUNK
            pltpu.make_async_copy(x_hbm.at[pl.ds(row0, CHUNK)], rb, sr).wait()
            pltpu.make_async_copy(idx_hbm.at[pl.ds(row0, CHUNK)], ib, si).wait()

        def start_scatter(b):
            rb, ib, _, _, ss = bufs[b]
            pltpu.async_copy(rb, acc.at[ib], ss, add=True)

        def wait_scatter(b):
            rb, ib, _, _, ss = bufs[b]
            pltpu.make_async_copy(rb, acc.at[ib], ss).wait()

        load_chunk(0, 1)
        load_chunk(1, 2)

        zeros16 = jnp.zeros((16,), jnp.float32)

        def zero_body(i, _):
            for j in range(D // 16):
                rows_0[i, pl.ds(j * 16, 16)] = zeros16
            return 0

        lax.fori_loop(0, CHUNK, zero_body, 0)

        strip = s * STRIP
        rem = STRIP - (STRIP // CHUNK) * CHUNK
        zero_copies = [
            (rows_0, acc.at[pl.ds(strip + kk * CHUNK, CHUNK)])
            for kk in range(STRIP // CHUNK)
        ]
        zero_copies.append(
            (rows_0.at[pl.ds(0, rem)],
             acc.at[pl.ds(strip + (STRIP // CHUNK) * CHUNK, rem)])
        )
        for src, dst in zero_copies:
            pltpu.async_copy(src, dst, ss0)
        for src, dst in zero_copies:
            pltpu.make_async_copy(src, dst, ss0).wait()

        @pl.when(s == NUM_SUBCORES - 1)
        def _zero_last():
            pltpu.sync_copy(
                rows_0.at[pl.ds(0, STRIP_REM)],
                acc.at[pl.ds(STRIP_COVERED, STRIP_REM)],
            )

        plsc.subcore_barrier()

        wait_load(0, 1)
        load_chunk(2, 0)
        start_scatter(1)

        def triple_body(t, _):
            for off, b in ((1, 2), (2, 0), (3, 1)):
                i = 3 * t + off
                wait_load(i, b)
                wait_scatter((b + 2) % 3)
                load_chunk(i + 2, (b + 2) % 3)
                start_scatter(b)
            return 0

        lax.fori_loop(0, (NUM_FULL_CHUNKS - 3) // 3, triple_body, 0)

        rowt = base + NUM_FULL_CHUNKS * CHUNK
        wait_load(NUM_FULL_CHUNKS - 2, 2)
        wait_scatter(1)
        pltpu.async_copy(x_hbm.at[pl.ds(rowt, TAIL)], rows_1.at[pl.ds(0, TAIL)], sr1)
        pltpu.async_copy(idx_hbm.at[pl.ds(rowt, TAIL)], idx_t, si1)
        start_scatter(2)
        wait_load(NUM_FULL_CHUNKS - 1, 0)
        wait_scatter(2)
        start_scatter(0)
        wait_scatter(0)

        pltpu.make_async_copy(
            x_hbm.at[pl.ds(rowt, TAIL)], rows_1.at[pl.ds(0, TAIL)], sr1
        ).wait()
        pltpu.make_async_copy(idx_hbm.at[pl.ds(rowt, TAIL)], idx_t, si1).wait()
        pltpu.sync_copy(rows_1.at[pl.ds(0, TAIL)], acc.at[idx_t], add=True)

        plsc.subcore_barrier()

        pltpu.sync_copy(
            acc.at[pl.ds(strip, STRIP)],
            out_hbm.at[c, pl.ds(strip, STRIP)],
        )

        @pl.when(s == NUM_SUBCORES - 1)
        def _dump_last():
            pltpu.sync_copy(
                acc.at[pl.ds(STRIP_COVERED, STRIP_REM)],
                out_hbm.at[c, pl.ds(STRIP_COVERED, STRIP_REM)],
            )

    return k(x, batch_index)


def _combine_body(p_ref, o_ref):
    o_ref[...] = p_ref[0] + p_ref[1]


def _combine(partials):
    blk = 1000
    return pl.pallas_call(
        _combine_body,
        grid=(NUM_SEGMENTS // blk,),
        in_specs=[pl.BlockSpec((NUM_CORES, blk, D), lambda i: (0, i, 0))],
        out_specs=pl.BlockSpec((blk, D), lambda i: (i, 0)),
        out_shape=jax.ShapeDtypeStruct((NUM_SEGMENTS, D), jnp.float32),
    )(partials)


def kernel(x, batch_index):
    idx = batch_index.astype(jnp.int32)
    partials = _seg_sum_sc(x, idx)
    return _combine(partials)


# speedup vs baseline: 1.0215x; 1.0215x over previous
import functools

import jax
import jax.numpy as jnp
from jax import lax
from jax.experimental import pallas as pl
from jax.experimental.pallas import tpu as pltpu
from jax.experimental.pallas import tpu_sc as plsc

N = 320000
D = 128
NUM_SEGMENTS = 10000

NUM_CORES = 2
NUM_SUBCORES = 16
NUM_WORKERS = NUM_CORES * NUM_SUBCORES

ROWS_PER_WORKER = N // NUM_WORKERS
CHUNK = 128
NUM_FULL_CHUNKS = ROWS_PER_WORKER // CHUNK
TAIL = ROWS_PER_WORKER - NUM_FULL_CHUNKS * CHUNK
STRIP = 624
STRIP_COVERED = STRIP * NUM_SUBCORES
STRIP_REM = NUM_SEGMENTS - STRIP_COVERED


def _seg_sum_sc(x, batch_index):
    mesh = plsc.VectorSubcoreMesh(core_axis_name="c", subcore_axis_name="s")

    @functools.partial(
        pl.kernel,
        mesh=mesh,
        out_type=jax.ShapeDtypeStruct((NUM_CORES, NUM_SEGMENTS, D), jnp.float32),
        scratch_types=[
            pltpu.VMEM_SHARED((NUM_SEGMENTS, D), jnp.float32),
            pltpu.VMEM((CHUNK, D), jnp.float32),
            pltpu.VMEM((CHUNK,), jnp.int32),
            pltpu.VMEM((CHUNK, D), jnp.float32),
            pltpu.VMEM((CHUNK,), jnp.int32),
            pltpu.VMEM((CHUNK, D), jnp.float32),
            pltpu.VMEM((CHUNK,), jnp.int32),
            pltpu.VMEM((TAIL,), jnp.int32),
            pltpu.SemaphoreType.DMA,
            pltpu.SemaphoreType.DMA,
            pltpu.SemaphoreType.DMA,
            pltpu.SemaphoreType.DMA,
            pltpu.SemaphoreType.DMA,
            pltpu.SemaphoreType.DMA,
            pltpu.SemaphoreType.DMA,
            pltpu.SemaphoreType.DMA,
            pltpu.SemaphoreType.DMA,
        ],
    )
    def k(x_hbm, idx_hbm, out_hbm, acc, rows_0, idx_0, rows_1, idx_1,
          rows_2, idx_2, idx_t, sr0, si0, sr1, si1, sr2, si2,
          ss0, ss1, ss2):
        c = lax.axis_index("c")
        s = lax.axis_index("s")
        wid = c * NUM_SUBCORES + s
        base = wid * ROWS_PER_WORKER

        bufs = (
            (rows_0, idx_0, sr0, si0, ss0),
            (rows_1, idx_1, sr1, si1, ss1),
            (rows_2, idx_2, sr2, si2, ss2),
        )

        def load_chunk(i, b):
            rb, ib, sr, si, _ = bufs[b]
            row0 = base + i * CHUNK
            pltpu.async_copy(x_hbm.at[pl.ds(row0, CHUNK)], rb, sr)
            pltpu.async_copy(idx_hbm.at[pl.ds(row0, CHUNK)], ib, si)

        def wait_load(i, b):
            rb, ib, sr, si, _ = bufs[b]
            row0 = base + i * CHUNK
            pltpu.make_async_copy(x_hbm.at[pl.ds(row0, CHUNK)], rb, sr).wait()
            pltpu.make_async_copy(idx_hbm.at[pl.ds(row0, CHUNK)], ib, si).wait()

        def start_scatter(b):
            rb, ib, _, _, ss = bufs[b]
            pltpu.async_copy(rb, acc.at[ib], ss, add=True)

        def wait_scatter(b):
            rb, ib, _, _, ss = bufs[b]
            pltpu.make_async_copy(rb, acc.at[ib], ss).wait()

        load_chunk(0, 1)
        load_chunk(1, 2)

        zeros16 = jnp.zeros((16,), jnp.float32)

        def zero_body(i, _):
            for j in range(D // 16):
                rows_0[i, pl.ds(j * 16, 16)] = zeros16
            return 0

        lax.fori_loop(0, CHUNK, zero_body, 0)

        strip = s * STRIP
        rem = STRIP - (STRIP // CHUNK) * CHUNK
        zero_copies = [
            (rows_0, acc.at[pl.ds(strip + kk * CHUNK, CHUNK)])
            for kk in range(STRIP // CHUNK)
        ]
        zero_copies.append(
            (rows_0.at[pl.ds(0, rem)],
             acc.at[pl.ds(strip + (STRIP // CHUNK) * CHUNK, rem)])
        )
        for src, dst in zero_copies:
            pltpu.async_copy(src, dst, ss0)
        for src, dst in zero_copies:
            pltpu.make_async_copy(src, dst, ss0).wait()

        @pl.when(s == NUM_SUBCORES - 1)
        def _zero_last():
            pltpu.sync_copy(
                rows_0.at[pl.ds(0, STRIP_REM)],
                acc.at[pl.ds(STRIP_COVERED, STRIP_REM)],
            )

        plsc.subcore_barrier()

        wait_load(0, 1)
        load_chunk(2, 0)
        start_scatter(1)

        def triple_body(t, _):
            for off, b in ((1, 2), (2, 0), (3, 1)):
                i = 3 * t + off
                wait_load(i, b)
                wait_scatter((b + 2) % 3)
                load_chunk(i + 2, (b + 2) % 3)
                start_scatter(b)
            return 0

        lax.fori_loop(0, (NUM_FULL_CHUNKS - 3) // 3, triple_body, 0)

        rowt = base + NUM_FULL_CHUNKS * CHUNK
        wait_load(NUM_FULL_CHUNKS - 2, 2)
        wait_scatter(1)
        pltpu.async_copy(x_hbm.at[pl.ds(rowt, TAIL)], rows_1.at[pl.ds(0, TAIL)], sr1)
        pltpu.async_copy(idx_hbm.at[pl.ds(rowt, TAIL)], idx_t, si1)
        start_scatter(2)
        wait_load(NUM_FULL_CHUNKS - 1, 0)
        wait_scatter(2)
        start_scatter(0)
        wait_scatter(0)

        pltpu.make_async_copy(
            x_hbm.at[pl.ds(rowt, TAIL)], rows_1.at[pl.ds(0, TAIL)], sr1
        ).wait()
        pltpu.make_async_copy(idx_hbm.at[pl.ds(rowt, TAIL)], idx_t, si1).wait()
        pltpu.sync_copy(rows_1.at[pl.ds(0, TAIL)], acc.at[idx_t], add=True)

        plsc.subcore_barrier()

        pltpu.sync_copy(
            acc.at[pl.ds(strip, STRIP)],
            out_hbm.at[c, pl.ds(strip, STRIP)],
        )

        @pl.when(s == NUM_SUBCORES - 1)
        def _dump_last():
            pltpu.sync_copy(
                acc.at[pl.ds(STRIP_COVERED, STRIP_REM)],
                out_hbm.at[c, pl.ds(STRIP_COVERED, STRIP_REM)],
            )

    return k(x, batch_index)


def _combine_body(p_ref, o_ref):
    o_ref[...] = p_ref[0] + p_ref[1]


def _combine(partials):
    blk = 2000
    return pl.pallas_call(
        _combine_body,
        grid=(NUM_SEGMENTS // blk,),
        in_specs=[pl.BlockSpec((NUM_CORES, blk, D), lambda i: (0, i, 0))],
        out_specs=pl.BlockSpec((blk, D), lambda i: (i, 0)),
        out_shape=jax.ShapeDtypeStruct((NUM_SEGMENTS, D), jnp.float32),
    )(partials)


def kernel(x, batch_index):
    idx = batch_index.astype(jnp.int32)
    partials = _seg_sum_sc(x, idx)
    return _combine(partials)


# speedup vs baseline: 1.0364x; 1.0146x over previous
import functools

import jax
import jax.numpy as jnp
from jax import lax
from jax.experimental import pallas as pl
from jax.experimental.pallas import tpu as pltpu
from jax.experimental.pallas import tpu_sc as plsc

N = 320000
D = 128
NUM_SEGMENTS = 10000

NUM_CORES = 2
NUM_SUBCORES = 16
NUM_WORKERS = NUM_CORES * NUM_SUBCORES

ROWS_PER_WORKER = N // NUM_WORKERS
CHUNK = 128
NUM_FULL_CHUNKS = ROWS_PER_WORKER // CHUNK
TAIL = ROWS_PER_WORKER - NUM_FULL_CHUNKS * CHUNK
STRIP = 624
STRIP_COVERED = STRIP * NUM_SUBCORES
STRIP_REM = NUM_SEGMENTS - STRIP_COVERED


def _seg_sum_sc(x, batch_index):
    mesh = plsc.VectorSubcoreMesh(core_axis_name="c", subcore_axis_name="s")

    @functools.partial(
        pl.kernel,
        mesh=mesh,
        out_type=jax.ShapeDtypeStruct((NUM_CORES, NUM_SEGMENTS, D), jnp.float32),
        scratch_types=[
            pltpu.VMEM_SHARED((NUM_SEGMENTS, D), jnp.float32),
            pltpu.VMEM((CHUNK, D), jnp.float32),
            pltpu.VMEM((CHUNK,), jnp.int32),
            pltpu.VMEM((CHUNK, D), jnp.float32),
            pltpu.VMEM((CHUNK,), jnp.int32),
            pltpu.VMEM((CHUNK, D), jnp.float32),
            pltpu.VMEM((CHUNK,), jnp.int32),
            pltpu.VMEM((TAIL,), jnp.int32),
            pltpu.SemaphoreType.DMA,
            pltpu.SemaphoreType.DMA,
            pltpu.SemaphoreType.DMA,
            pltpu.SemaphoreType.DMA,
            pltpu.SemaphoreType.DMA,
            pltpu.SemaphoreType.DMA,
            pltpu.SemaphoreType.DMA,
            pltpu.SemaphoreType.DMA,
            pltpu.SemaphoreType.DMA,
        ],
    )
    def k(x_hbm, idx_hbm, out_hbm, acc, rows_0, idx_0, rows_1, idx_1,
          rows_2, idx_2, idx_t, sr0, si0, sr1, si1, sr2, si2,
          ss0, ss1, ss2):
        c = lax.axis_index("c")
        s = lax.axis_index("s")
        wid = c * NUM_SUBCORES + s
        base = wid * ROWS_PER_WORKER

        bufs = (
            (rows_0, idx_0, sr0, si0, ss0),
            (rows_1, idx_1, sr1, si1, ss1),
            (rows_2, idx_2, sr2, si2, ss2),
        )

        def load_chunk(i, b):
            rb, ib, sr, si, _ = bufs[b]
            row0 = base + i * CHUNK
            pltpu.async_copy(x_hbm.at[pl.ds(row0, CHUNK)], rb, sr)
            pltpu.async_copy(idx_hbm.at[pl.ds(row0, CHUNK)], ib, si)

        def wait_load(i, b):
            rb, ib, sr, si, _ = bufs[b]
            row0 = base + i * CHUNK
            pltpu.make_async_copy(x_hbm.at[pl.ds(row0, CHUNK)], rb, sr).wait()
            pltpu.make_async_copy(idx_hbm.at[pl.ds(row0, CHUNK)], ib, si).wait()

        def start_scatter(b):
            rb, ib, _, _, ss = bufs[b]
            pltpu.async_copy(rb, acc.at[ib], ss, add=True)

        def wait_scatter(b):
            rb, ib, _, _, ss = bufs[b]
            pltpu.make_async_copy(rb, acc.at[ib], ss).wait()

        load_chunk(0, 1)
        load_chunk(1, 2)

        zeros16 = jnp.zeros((16,), jnp.float32)

        def zero_body(i, _):
            for j in range(D // 16):
                rows_0[i, pl.ds(j * 16, 16)] = zeros16
            return 0

        lax.fori_loop(0, CHUNK, zero_body, 0)

        strip = s * STRIP
        rem = STRIP - (STRIP // CHUNK) * CHUNK
        zero_copies = [
            (rows_0, acc.at[pl.ds(strip + kk * CHUNK, CHUNK)])
            for kk in range(STRIP // CHUNK)
        ]
        zero_copies.append(
            (rows_0.at[pl.ds(0, rem)],
             acc.at[pl.ds(strip + (STRIP // CHUNK) * CHUNK, rem)])
        )
        for src, dst in zero_copies:
            pltpu.async_copy(src, dst, ss0)
        for src, dst in zero_copies:
            pltpu.make_async_copy(src, dst, ss0).wait()

        @pl.when(s == NUM_SUBCORES - 1)
        def _zero_last():
            pltpu.sync_copy(
                rows_0.at[pl.ds(0, STRIP_REM)],
                acc.at[pl.ds(STRIP_COVERED, STRIP_REM)],
            )

        plsc.subcore_barrier()

        wait_load(0, 1)
        load_chunk(2, 0)
        start_scatter(1)

        def triple_body(t, _):
            for off, b in ((1, 2), (2, 0), (3, 1)):
                i = 3 * t + off
                wait_load(i, b)
                wait_scatter((b + 2) % 3)
                load_chunk(i + 2, (b + 2) % 3)
                start_scatter(b)
            return 0

        lax.fori_loop(0, (NUM_FULL_CHUNKS - 3) // 3, triple_body, 0)

        rowt = base + NUM_FULL_CHUNKS * CHUNK
        wait_load(NUM_FULL_CHUNKS - 2, 2)
        wait_scatter(1)
        pltpu.async_copy(x_hbm.at[pl.ds(rowt, TAIL)], rows_1.at[pl.ds(0, TAIL)], sr1)
        pltpu.async_copy(idx_hbm.at[pl.ds(rowt, TAIL)], idx_t, si1)
        start_scatter(2)
        wait_load(NUM_FULL_CHUNKS - 1, 0)
        wait_scatter(2)
        start_scatter(0)
        wait_scatter(0)

        pltpu.make_async_copy(
            x_hbm.at[pl.ds(rowt, TAIL)], rows_1.at[pl.ds(0, TAIL)], sr1
        ).wait()
        pltpu.make_async_copy(idx_hbm.at[pl.ds(rowt, TAIL)], idx_t, si1).wait()
        pltpu.sync_copy(rows_1.at[pl.ds(0, TAIL)], acc.at[idx_t], add=True)

        plsc.subcore_barrier()

        pltpu.sync_copy(
            acc.at[pl.ds(strip, STRIP)],
            out_hbm.at[c, pl.ds(strip, STRIP)],
        )

        @pl.when(s == NUM_SUBCORES - 1)
        def _dump_last():
            pltpu.sync_copy(
                acc.at[pl.ds(STRIP_COVERED, STRIP_REM)],
                out_hbm.at[c, pl.ds(STRIP_COVERED, STRIP_REM)],
            )

    return k(x, batch_index)


def _combine_body(p_ref, o_ref):
    o_ref[...] = p_ref[0] + p_ref[1]


def _combine(partials):
    blk = 5000
    return pl.pallas_call(
        _combine_body,
        grid=(NUM_SEGMENTS // blk,),
        in_specs=[pl.BlockSpec((NUM_CORES, blk, D), lambda i: (0, i, 0))],
        out_specs=pl.BlockSpec((blk, D), lambda i: (i, 0)),
        out_shape=jax.ShapeDtypeStruct((NUM_SEGMENTS, D), jnp.float32),
    )(partials)


def kernel(x, batch_index):
    idx = batch_index.astype(jnp.int32)
    partials = _seg_sum_sc(x, idx)
    return _combine(partials)
